# baseline (speedup 1.0000x reference)
import functools

import jax
import jax.numpy as jnp
from jax import lax
from jax.experimental import pallas as pl
from jax.experimental.pallas import tpu as pltpu
from jax.experimental.pallas import tpu_sc as plsc

_NC = 2
_NS = 16
_NW = _NC * _NS
_CW = 128


def _node_body(x_ref, attnw_ref, attnb_ref, anchor_ref, wsrc_ref, wdst_ref,
               out_ref, psrc_ref, pdst_ref):
    xb = x_ref[...]
    s = lax.dot_general(
        xb, attnw_ref[...], (((1,), (1,)), ((), ())),
        preferred_element_type=jnp.float32) + attnb_ref[...]
    s = s - jnp.max(s, axis=1, keepdims=True)
    e = jnp.exp(s)
    w = e / jnp.sum(e, axis=1, keepdims=True)
    out_ref[...] = xb + lax.dot_general(
        w, anchor_ref[...], (((1,), (0,)), ((), ())),
        preferred_element_type=jnp.float32)
    psrc_ref[...] = lax.dot_general(
        xb, wsrc_ref[...], (((1,), (1,)), ((), ())),
        preferred_element_type=jnp.float32)
    pdst_ref[...] = lax.dot_general(
        xb, wdst_ref[...], (((1,), (1,)), ((), ())),
        preferred_element_type=jnp.float32)


def _edge_body(lg_ref, wb_ref, anchor_ref, out_ref):
    l = lg_ref[...] + wb_ref[...]
    l = jnp.where(l >= 0, l, 0.01 * l)
    l = l - jnp.max(l, axis=1, keepdims=True)
    e = jnp.exp(l)
    b = (e / jnp.sum(e, axis=1, keepdims=True)).astype(jnp.bfloat16)
    out_ref[...] = lax.dot_general(
        b, anchor_ref[...].astype(jnp.bfloat16), (((1,), (0,)), ((), ())),
        preferred_element_type=jnp.float32)


_NB = 20


def _sc_gather_body(nchunk, cw_out, ptab_s, ptab_d, src_hbm, dst_hbm,
                    out_hbm, sidx, didx, rows, sem_g, sem_a, sem_w):
    wid = lax.axis_index("s") * _NC + lax.axis_index("c")
    pltpu.sync_copy(src_hbm.at[wid], sidx)
    pltpu.sync_copy(dst_hbm.at[wid], didx)

    def wave(g, carry):
        gs = []
        for b in range(_NB):
            j = g * _NB + b
            gs.append(
                pltpu.async_copy(ptab_s.at[sidx.at[j]], rows.at[b], sem_g))
        ads = []
        for b in range(_NB):
            gs[b].wait()
            j = g * _NB + b
            ads.append(
                pltpu.async_copy(ptab_d.at[didx.at[j]], rows.at[b], sem_a,
                                 add=True))
        ws = []
        for b in range(_NB):
            ads[b].wait()
            j = g * _NB + b
            ws.append(
                pltpu.async_copy(rows.at[b, pl.ds(0, cw_out)],
                                 out_hbm.at[wid, j], sem_w))
        for b in range(_NB):
            ws[b].wait()
        return carry

    lax.fori_loop(0, nchunk // _NB, wave, 0, unroll=False)


def _sc_gather(ptab_s, ptab_d, srcp, dstp, nchunk, cw_out):
    mesh = plsc.VectorSubcoreMesh(
        core_axis_name="c", subcore_axis_name="s",
        num_cores=_NC, num_subcores=_NS)
    fn = pl.kernel(
        functools.partial(_sc_gather_body, nchunk, cw_out),
        out_type=jax.ShapeDtypeStruct((_NW, nchunk, cw_out, 16),
                                      jnp.float32),
        mesh=mesh,
        scratch_types=[
            pltpu.VMEM((nchunk, _CW), jnp.int32),
            pltpu.VMEM((nchunk, _CW), jnp.int32),
            pltpu.VMEM((_NB, _CW, 16), jnp.float32),
            pltpu.SemaphoreType.DMA,
            pltpu.SemaphoreType.DMA,
            pltpu.SemaphoreType.DMA,
        ],
        compiler_params=pltpu.CompilerParams(use_tc_tiling_on_sc=False),
    )
    return fn(ptab_s, ptab_d, srcp, dstp)


def kernel(x, edge_index, layer, node_anchor, attn_W, attn_b, edge_anchor,
           w_W, w_b):
    n, d = x.shape
    a = node_anchor.shape[0]
    e = edge_index.shape[1]

    w_src = w_W[:, :d]
    w_dst = w_W[:, d:]
    attn_b2 = attn_b.reshape(1, a)
    w_b2 = w_b.reshape(1, a)

    bn = 2000
    grid_n = n // bn
    node_prompted_x, psrc, pdst = pl.pallas_call(
        _node_body,
        grid=(grid_n,),
        in_specs=[
            pl.BlockSpec((bn, d), lambda i: (i, 0)),
            pl.BlockSpec((a, d), lambda i: (0, 0)),
            pl.BlockSpec((1, a), lambda i: (0, 0)),
            pl.BlockSpec((a, d), lambda i: (0, 0)),
            pl.BlockSpec((a, d), lambda i: (0, 0)),
            pl.BlockSpec((a, d), lambda i: (0, 0)),
        ],
        out_specs=[
            pl.BlockSpec((bn, d), lambda i: (i, 0)),
            pl.BlockSpec((bn, a), lambda i: (i, 0)),
            pl.BlockSpec((bn, a), lambda i: (i, 0)),
        ],
        out_shape=[
            jax.ShapeDtypeStruct((n, d), jnp.float32),
            jax.ShapeDtypeStruct((n, a), jnp.float32),
            jax.ShapeDtypeStruct((n, a), jnp.float32),
        ],
    )(x, attn_W, attn_b2, node_anchor, w_src, w_dst)

    cw_out = 125
    nchunk = e // (_NW * cw_out)
    src = edge_index[0].astype(jnp.int32).reshape(_NW, nchunk, cw_out)
    dst = edge_index[1].astype(jnp.int32).reshape(_NW, nchunk, cw_out)
    padw = ((0, 0), (0, 0), (0, _CW - cw_out))
    srcp = jnp.pad(src, padw)
    dstp = jnp.pad(dst, padw)
    logits = _sc_gather(psrc, pdst, srcp, dstp, nchunk, cw_out).reshape(e, a)

    be = 8000
    grid_e = e // be
    edge_prompt = pl.pallas_call(
        _edge_body,
        grid=(grid_e,),
        in_specs=[
            pl.BlockSpec((be, a), lambda i: (i, 0)),
            pl.BlockSpec((1, a), lambda i: (0, 0)),
            pl.BlockSpec((a, d), lambda i: (0, 0)),
        ],
        out_specs=pl.BlockSpec((be, d), lambda i: (i, 0)),
        out_shape=jax.ShapeDtypeStruct((e, d), jnp.float32),
    )(logits, w_b2, edge_anchor)

    return (node_prompted_x, edge_prompt)


# speedup vs baseline: 1.0186x; 1.0186x over previous
import functools

import jax
import jax.numpy as jnp
from jax import lax
from jax.experimental import pallas as pl
from jax.experimental.pallas import tpu as pltpu
from jax.experimental.pallas import tpu_sc as plsc

_NC = 2
_NS = 16
_NW = _NC * _NS
_CW = 128
_NB = 20


def _node_body(x_ref, attnw_ref, attnb_ref, anchor_ref, wsrc_ref, wdst_ref,
               out_ref, psrc_ref, pdst_ref):
    xb = x_ref[...]
    s = lax.dot_general(
        xb, attnw_ref[...], (((1,), (1,)), ((), ())),
        preferred_element_type=jnp.float32) + attnb_ref[...]
    s = s - jnp.max(s, axis=1, keepdims=True)
    e = jnp.exp(s)
    w = e / jnp.sum(e, axis=1, keepdims=True)
    out_ref[...] = xb + lax.dot_general(
        w, anchor_ref[...], (((1,), (0,)), ((), ())),
        preferred_element_type=jnp.float32)
    psrc_ref[...] = lax.dot_general(
        xb, wsrc_ref[...], (((1,), (1,)), ((), ())),
        preferred_element_type=jnp.float32)
    pdst_ref[...] = lax.dot_general(
        xb, wdst_ref[...], (((1,), (1,)), ((), ())),
        preferred_element_type=jnp.float32)


def _edge_body(lg_ref, wb_ref, anchor_ref, out_ref):
    l = lg_ref[...] + wb_ref[...]
    l = jnp.where(l >= 0, l, 0.01 * l)
    l = l - jnp.max(l, axis=1, keepdims=True)
    e = jnp.exp(l)
    b = (e / jnp.sum(e, axis=1, keepdims=True)).astype(jnp.bfloat16)
    out_ref[...] = lax.dot_general(
        b, anchor_ref[...].astype(jnp.bfloat16), (((1,), (0,)), ((), ())),
        preferred_element_type=jnp.float32)


def _edge_body_alias(lg_ref, wb_ref, anchor_ref, prev_ref, out_ref):
    del prev_ref
    _edge_body(lg_ref, wb_ref, anchor_ref, out_ref)


def _sc_gather_body(nchunk, cw_out, ptab_s, ptab_d, src_hbm, dst_hbm,
                    out_hbm, sidx, didx, rows, sem_g, sem_a, sem_w):
    wid = lax.axis_index("s") * _NC + lax.axis_index("c")
    pltpu.sync_copy(src_hbm.at[wid], sidx)
    pltpu.sync_copy(dst_hbm.at[wid], didx)

    def wave(g, carry):
        gs = []
        for b in range(_NB):
            j = g * _NB + b
            gs.append(
                pltpu.async_copy(ptab_s.at[sidx.at[j]], rows.at[b], sem_g))
        ads = []
        for b in range(_NB):
            gs[b].wait()
            j = g * _NB + b
            ads.append(
                pltpu.async_copy(ptab_d.at[didx.at[j]], rows.at[b], sem_a,
                                 add=True))
        ws = []
        for b in range(_NB):
            ads[b].wait()
            j = g * _NB + b
            ws.append(
                pltpu.async_copy(rows.at[b, pl.ds(0, cw_out)],
                                 out_hbm.at[wid, j], sem_w))
        for b in range(_NB):
            ws[b].wait()
        return carry

    lax.fori_loop(0, nchunk // _NB, wave, 0, unroll=False)


def _sc_gather(ptab_s, ptab_d, srcp, dstp, nchunk, cw_out):
    mesh = plsc.VectorSubcoreMesh(
        core_axis_name="c", subcore_axis_name="s",
        num_cores=_NC, num_subcores=_NS)
    fn = pl.kernel(
        functools.partial(_sc_gather_body, nchunk, cw_out),
        out_type=jax.ShapeDtypeStruct((_NW, nchunk, cw_out, 16),
                                      jnp.float32),
        mesh=mesh,
        scratch_types=[
            pltpu.VMEM((nchunk, _CW), jnp.int32),
            pltpu.VMEM((nchunk, _CW), jnp.int32),
            pltpu.VMEM((_NB, _CW, 16), jnp.float32),
            pltpu.SemaphoreType.DMA,
            pltpu.SemaphoreType.DMA,
            pltpu.SemaphoreType.DMA,
        ],
        compiler_params=pltpu.CompilerParams(use_tc_tiling_on_sc=False),
    )
    return fn(ptab_s, ptab_d, srcp, dstp)


def kernel(x, edge_index, layer, node_anchor, attn_W, attn_b, edge_anchor,
           w_W, w_b):
    n, d = x.shape
    a = node_anchor.shape[0]
    e = edge_index.shape[1]

    w_src = w_W[:, :d]
    w_dst = w_W[:, d:]
    attn_b2 = attn_b.reshape(1, a)
    w_b2 = w_b.reshape(1, a)

    bn = 2000
    grid_n = n // bn
    node_prompted_x, psrc, pdst = pl.pallas_call(
        _node_body,
        grid=(grid_n,),
        in_specs=[
            pl.BlockSpec((bn, d), lambda i: (i, 0)),
            pl.BlockSpec((a, d), lambda i: (0, 0)),
            pl.BlockSpec((1, a), lambda i: (0, 0)),
            pl.BlockSpec((a, d), lambda i: (0, 0)),
            pl.BlockSpec((a, d), lambda i: (0, 0)),
            pl.BlockSpec((a, d), lambda i: (0, 0)),
        ],
        out_specs=[
            pl.BlockSpec((bn, d), lambda i: (i, 0)),
            pl.BlockSpec((bn, a), lambda i: (i, 0)),
            pl.BlockSpec((bn, a), lambda i: (i, 0)),
        ],
        out_shape=[
            jax.ShapeDtypeStruct((n, d), jnp.float32),
            jax.ShapeDtypeStruct((n, a), jnp.float32),
            jax.ShapeDtypeStruct((n, a), jnp.float32),
        ],
    )(x, attn_W, attn_b2, node_anchor, w_src, w_dst)

    cw_out = 125
    eh = e // 2
    nchunk = eh // (_NW * cw_out)
    src = edge_index[0].astype(jnp.int32)
    dst = edge_index[1].astype(jnp.int32)
    padw = ((0, 0), (0, 0), (0, _CW - cw_out))
    logits = []
    for h in range(2):
        s_h = src[h * eh:(h + 1) * eh].reshape(_NW, nchunk, cw_out)
        d_h = dst[h * eh:(h + 1) * eh].reshape(_NW, nchunk, cw_out)
        lg = _sc_gather(psrc, pdst, jnp.pad(s_h, padw), jnp.pad(d_h, padw),
                        nchunk, cw_out).reshape(eh, a)
        logits.append(lg)

    be = 8000
    grid_h = eh // be
    out1 = pl.pallas_call(
        _edge_body,
        grid=(grid_h,),
        in_specs=[
            pl.BlockSpec((be, a), lambda i: (i, 0)),
            pl.BlockSpec((1, a), lambda i: (0, 0)),
            pl.BlockSpec((a, d), lambda i: (0, 0)),
        ],
        out_specs=pl.BlockSpec((be, d), lambda i: (i, 0)),
        out_shape=jax.ShapeDtypeStruct((e, d), jnp.float32),
    )(logits[0], w_b2, edge_anchor)
    edge_prompt = pl.pallas_call(
        _edge_body_alias,
        grid=(grid_h,),
        in_specs=[
            pl.BlockSpec((be, a), lambda i: (i, 0)),
            pl.BlockSpec((1, a), lambda i: (0, 0)),
            pl.BlockSpec((a, d), lambda i: (0, 0)),
            pl.BlockSpec(memory_space=pl.ANY),
        ],
        out_specs=pl.BlockSpec((be, d), lambda i: (i + grid_h, 0)),
        out_shape=jax.ShapeDtypeStruct((e, d), jnp.float32),
        input_output_aliases={3: 0},
    )(logits[1], w_b2, edge_anchor, out1)

    return (node_prompted_x, edge_prompt)
